# baseline (speedup 1.0000x reference)
import functools

import jax
import jax.numpy as jnp
from jax import lax
from jax.experimental import pallas as pl
from jax.experimental.pallas import tpu as pltpu
from jax.experimental.pallas import tpu_sc as plsc

D_MODEL_K = 64
MAX_LEN_K = 13941
ROWS_PAD = 13952

NC = 2
NS = 16
NW = NC * NS
CHUNK = 512


def _blur_body(t_ref, o_ref):
    x = t_ref[...]
    xm1 = jnp.concatenate([x[:1], x[:-1]], axis=0)
    xp1 = jnp.concatenate([x[1:], x[-1:]], axis=0)
    o_ref[...] = 0.5 * xm1 + x + 0.5 * xp1


def _blur(tpad):
    shp = jax.ShapeDtypeStruct((ROWS_PAD, D_MODEL_K), jnp.float32)
    return pl.pallas_call(_blur_body, out_shape=shp)(tpad)


def _make_gather(n):
    b_per_w = n // NW
    n_chunks = b_per_w // CHUNK
    mesh = plsc.VectorSubcoreMesh(core_axis_name="c", subcore_axis_name="s")

    @functools.partial(
        pl.kernel,
        mesh=mesh,
        compiler_params=pltpu.CompilerParams(use_tc_tiling_on_sc=False),
        out_type=jax.ShapeDtypeStruct((n, D_MODEL_K), jnp.float32),
        scratch_types=[
            pltpu.VMEM((b_per_w,), jnp.int32),
            pltpu.VMEM((CHUNK, D_MODEL_K), jnp.float32),
            pltpu.VMEM((CHUNK, D_MODEL_K), jnp.float32),
            pltpu.SemaphoreType.DMA,
            pltpu.SemaphoreType.DMA,
            pltpu.SemaphoreType.DMA,
            pltpu.SemaphoreType.DMA,
        ],
    )
    def gather_k(table_hbm, idx_hbm, out_hbm, idx_v, rows0, rows1, g0, g1, w0, w1):
        wid = lax.axis_index("s") * NC + lax.axis_index("c")
        base = wid * b_per_w
        rows = (rows0, rows1)
        gsem = (g0, g1)
        wsem = (w0, w1)

        def start_gather(i, b):
            pltpu.async_copy(
                table_hbm.at[idx_v.at[pl.ds(i * CHUNK, CHUNK)]], rows[b], gsem[b]
            )

        def wait_gather(b):
            pltpu.make_async_copy(
                table_hbm.at[pl.ds(0, CHUNK)], rows[b], gsem[b]
            ).wait()

        pltpu.sync_copy(idx_hbm.at[pl.ds(base, b_per_w)], idx_v)
        start_gather(0, 0)

        def slot(i, k, j):
            jn = 1 - j

            def prefetch():
                pltpu.make_async_copy(
                    rows[jn], out_hbm.at[pl.ds(base + (i - 1) * CHUNK, CHUNK)], wsem[jn]
                ).wait()
                start_gather(i + 1, jn)

            if j == 0:
                @pl.when(k >= 1)
                def _():
                    prefetch()

                @pl.when(k < 1)
                def _():
                    start_gather(i + 1, jn)
            else:
                @pl.when(k < n_chunks // 2 - 1)
                def _():
                    prefetch()

            wait_gather(j)
            pltpu.async_copy(
                rows[j], out_hbm.at[pl.ds(base + i * CHUNK, CHUNK)], wsem[j]
            )

        def outer(k, carry):
            slot(2 * k, k, 0)
            slot(2 * k + 1, k, 1)
            return carry

        lax.fori_loop(0, n_chunks // 2, outer, 0)

        for j in range(2):
            pltpu.make_async_copy(
                rows[j],
                out_hbm.at[pl.ds(base + (n_chunks - 2 + j) * CHUNK, CHUNK)],
                wsem[j],
            ).wait()

    return gather_k


def _transpose_body(in_ref, o_ref):
    x = in_ref[0]
    xt = jnp.transpose(x, (1, 0))
    o_ref[...] = xt.reshape(1, 8, 1, 8, 128)


def _make_transpose(n_b, n_s):
    sblk = n_b // 128
    grid = (n_s, sblk)
    return pl.pallas_call(
        _transpose_body,
        grid=grid,
        in_specs=[
            pl.BlockSpec((1, 128, D_MODEL_K), lambda t, sb: (t, sb, 0)),
        ],
        out_specs=pl.BlockSpec(
            (1, 8, 1, 8, 128), lambda t, sb: (t, 0, sb, 0, 0)
        ),
        out_shape=jax.ShapeDtypeStruct(
            (n_s, D_MODEL_K // 8, sblk, 8, 128), jnp.float32
        ),
    )


def kernel(pos, table):
    t = table.astype(jnp.float32)
    tpad = jnp.pad(t, ((0, ROWS_PAD - (MAX_LEN_K + 1)), (0, 0)))
    blurred = _blur(tpad)

    b, s = pos.shape
    idx_t = jnp.transpose(pos).astype(jnp.int32).reshape(-1)
    inter = _make_gather(b * s)(blurred, idx_t)
    out5 = _make_transpose(b, s)(inter.reshape(s, b, D_MODEL_K))
    return jnp.transpose(out5, (2, 4, 0, 1, 3)).reshape(b, s, D_MODEL_K)


# speedup vs baseline: 5.5709x; 5.5709x over previous
import functools

import jax
import jax.numpy as jnp
from jax import lax
from jax.experimental import pallas as pl
from jax.experimental.pallas import tpu as pltpu
from jax.experimental.pallas import tpu_sc as plsc

D_MODEL_K = 64
MAX_LEN_K = 13941
ROWS_PAD = 13952

NC = 2
NS = 16
NW = NC * NS
CHUNK = 512


def _blur_body(t_ref, o_ref):
    x = t_ref[...]
    xm1 = jnp.concatenate([x[:1], x[:-1]], axis=0)
    xp1 = jnp.concatenate([x[1:], x[-1:]], axis=0)
    o_ref[...] = 0.5 * xm1 + x + 0.5 * xp1


def _blur(tpad):
    shp = jax.ShapeDtypeStruct((ROWS_PAD, D_MODEL_K), jnp.float32)
    return pl.pallas_call(_blur_body, out_shape=shp)(tpad)


def _make_gather(n):
    b_per_w = n // NW
    n_chunks = b_per_w // CHUNK
    mesh = plsc.VectorSubcoreMesh(core_axis_name="c", subcore_axis_name="s")

    @functools.partial(
        pl.kernel,
        mesh=mesh,
        compiler_params=pltpu.CompilerParams(use_tc_tiling_on_sc=False),
        out_type=jax.ShapeDtypeStruct((n, D_MODEL_K), jnp.float32),
        scratch_types=[
            pltpu.VMEM((b_per_w,), jnp.int32),
            pltpu.VMEM((CHUNK, D_MODEL_K), jnp.float32),
            pltpu.VMEM((CHUNK, D_MODEL_K), jnp.float32),
            pltpu.SemaphoreType.DMA,
            pltpu.SemaphoreType.DMA,
            pltpu.SemaphoreType.DMA,
            pltpu.SemaphoreType.DMA,
        ],
    )
    def gather_k(table_hbm, idx_hbm, out_hbm, idx_v, rows0, rows1, g0, g1, w0, w1):
        wid = lax.axis_index("s") * NC + lax.axis_index("c")
        base = wid * b_per_w
        rows = (rows0, rows1)
        gsem = (g0, g1)
        wsem = (w0, w1)

        def start_gather(i, b):
            pltpu.async_copy(
                table_hbm.at[idx_v.at[pl.ds(i * CHUNK, CHUNK)]], rows[b], gsem[b]
            )

        def wait_gather(b):
            pltpu.make_async_copy(
                table_hbm.at[pl.ds(0, CHUNK)], rows[b], gsem[b]
            ).wait()

        pltpu.sync_copy(idx_hbm.at[pl.ds(base, b_per_w)], idx_v)
        start_gather(0, 0)

        def slot(i, k, j):
            jn = 1 - j

            def prefetch():
                pltpu.make_async_copy(
                    rows[jn], out_hbm.at[pl.ds(base + (i - 1) * CHUNK, CHUNK)], wsem[jn]
                ).wait()
                start_gather(i + 1, jn)

            if j == 0:
                @pl.when(k >= 1)
                def _():
                    prefetch()

                @pl.when(k < 1)
                def _():
                    start_gather(i + 1, jn)
            else:
                @pl.when(k < n_chunks // 2 - 1)
                def _():
                    prefetch()

            wait_gather(j)
            pltpu.async_copy(
                rows[j], out_hbm.at[pl.ds(base + i * CHUNK, CHUNK)], wsem[j]
            )

        def outer(k, carry):
            slot(2 * k, k, 0)
            slot(2 * k + 1, k, 1)
            return carry

        lax.fori_loop(0, n_chunks // 2, outer, 0)

        for j in range(2):
            pltpu.make_async_copy(
                rows[j],
                out_hbm.at[pl.ds(base + (n_chunks - 2 + j) * CHUNK, CHUNK)],
                wsem[j],
            ).wait()

    return gather_k


def kernel(pos, table):
    t = table.astype(jnp.float32)
    tpad = jnp.pad(t, ((0, ROWS_PAD - (MAX_LEN_K + 1)), (0, 0)))
    blurred = _blur(tpad)

    b, s = pos.shape
    idx = pos.astype(jnp.int32).reshape(-1)
    inter = _make_gather(b * s)(blurred, idx)
    return inter.reshape(b, s, D_MODEL_K)


# speedup vs baseline: 6.1919x; 1.1115x over previous
import functools

import jax
import jax.numpy as jnp
from jax import lax
from jax.experimental import pallas as pl
from jax.experimental.pallas import tpu as pltpu
from jax.experimental.pallas import tpu_sc as plsc

D_MODEL_K = 64
MAX_LEN_K = 13941
ROWS_PAD = 13952

NC = 2
NS = 16
NW = NC * NS


def _blur_body(t_ref, o_ref):
    x = t_ref[...]
    xm1 = jnp.concatenate([x[:1], x[:-1]], axis=0)
    xp1 = jnp.concatenate([x[1:], x[-1:]], axis=0)
    y = 0.5 * xm1 + x + 0.5 * xp1
    o_ref[...] = jnp.concatenate([y, y], axis=1)


def _blur(tpad):
    shp = jax.ShapeDtypeStruct((ROWS_PAD, 128), jnp.float32)
    return pl.pallas_call(_blur_body, out_shape=shp)(tpad)


def _make_gather(n_b, n_s):
    s_per_w = n_b // NW
    mesh = plsc.VectorSubcoreMesh(core_axis_name="c", subcore_axis_name="s")

    @functools.partial(
        pl.kernel,
        mesh=mesh,
        out_type=jax.ShapeDtypeStruct((n_b, n_s, D_MODEL_K), jnp.float32),
        scratch_types=[
            pltpu.VMEM((s_per_w * n_s,), jnp.int32),
            pltpu.VMEM((n_s, 128), jnp.float32),
            pltpu.VMEM((n_s, 128), jnp.float32),
            pltpu.VMEM((n_s, D_MODEL_K), jnp.float32),
            pltpu.VMEM((n_s, D_MODEL_K), jnp.float32),
            pltpu.SemaphoreType.DMA,
            pltpu.SemaphoreType.DMA,
            pltpu.SemaphoreType.DMA,
            pltpu.SemaphoreType.DMA,
        ],
    )
    def gather_k(
        table_hbm, idx_hbm, out_hbm, idx_v, rows0, rows1, cpt0, cpt1, g0, g1, w0, w1
    ):
        wid = lax.axis_index("s") * NC + lax.axis_index("c")
        base = wid * s_per_w
        rows = (rows0, rows1)
        cpts = (cpt0, cpt1)
        gsem = (g0, g1)
        wsem = (w0, w1)

        def start_gather(i, b):
            pltpu.async_copy(
                table_hbm.at[idx_v.at[pl.ds(i * n_s, n_s)]], rows[b], gsem[b]
            )

        def wait_gather(b):
            pltpu.make_async_copy(
                table_hbm.at[pl.ds(0, n_s)], rows[b], gsem[b]
            ).wait()

        def compact(b):
            def body(t8, carry):
                r0 = t8 * 8
                for rr in range(8):
                    for c4 in range(D_MODEL_K // 16):
                        cpts[b][r0 + rr, pl.ds(c4 * 16, 16)] = rows[b][
                            r0 + rr, pl.ds(c4 * 16, 16)
                        ]
                return carry

            lax.fori_loop(0, n_s // 8, body, 0)

        pltpu.sync_copy(idx_hbm.at[pl.ds(base * n_s, s_per_w * n_s)], idx_v)
        start_gather(0, 0)

        def slot(i, k, j):
            jn = 1 - j

            def prefetch():
                start_gather(i + 1, jn)

            if j == 0:
                prefetch()
            else:
                @pl.when(k < s_per_w // 2 - 1)
                def _():
                    prefetch()

            wait_gather(j)

            @pl.when(k >= 1)
            def _():
                pltpu.make_async_copy(
                    cpts[j], out_hbm.at[base + i - 2], wsem[j]
                ).wait()

            compact(j)
            pltpu.async_copy(cpts[j], out_hbm.at[base + i], wsem[j])

        def outer(k, carry):
            slot(2 * k, k, 0)
            slot(2 * k + 1, k, 1)
            return carry

        lax.fori_loop(0, s_per_w // 2, outer, 0)

        for j in range(2):
            pltpu.make_async_copy(
                cpts[j], out_hbm.at[base + s_per_w - 2 + j], wsem[j]
            ).wait()

    return gather_k


def kernel(pos, table):
    t = table.astype(jnp.float32)
    tpad = jnp.pad(t, ((0, ROWS_PAD - (MAX_LEN_K + 1)), (0, 0)))
    blurred = _blur(tpad)

    b, s = pos.shape
    idx = pos.astype(jnp.int32).reshape(-1)
    return _make_gather(b, s)(blurred, idx)
